# Initial kernel scaffold; baseline (speedup 1.0000x reference)
import jax
import jax.numpy as jnp
from jax.experimental import pallas as pl


def kernel(x_input, p_indices, p_values, f_indices, f_values, attn_w, attn_o_w, attn_norm_w, ffn_norm_w, ffn_up_w, ffn_down_w, p_ffn_experts, f_ffn_experts, p_token_keys, f_token_keys, p_token_router_bias, f_token_router_bias):
    raise NotImplementedError("write your pallas kernel here")


# speedup vs baseline: 2.2221x; 2.2221x over previous
import functools
import math

import jax
import jax.numpy as jnp
from jax.experimental import pallas as pl

DIM = 768
HEADS = 12
HDIM = 64
E = 8
TOPK = 2
EXP_DIM = 256
DIM_S = 2048
EPS = 1e-5
THETA = 10000.0
B = 2
S = 2048
N = B * S
L = 2 * S


_QKV_TILE = 256


def _qkv_kernel(x_ref, w_ref, nw_ref, cos_ref, sin_ref, out_ref):
    x = x_ref[...]
    xn = x * jax.lax.rsqrt(jnp.mean(x * x, axis=-1, keepdims=True) + EPS)
    xn = xn * nw_ref[...]
    qkv = jnp.dot(xn, w_ref[...], preferred_element_type=jnp.float32)
    cos = cos_ref[...][:, None, :]
    sin = sin_ref[...][:, None, :]

    def rot(v):
        v = v.reshape(_QKV_TILE, HEADS, HDIM)
        x1 = v[..., : HDIM // 2]
        x2 = v[..., HDIM // 2:]
        y1 = x1 * cos + x2 * sin
        y2 = -x1 * sin + x2 * cos
        return jnp.concatenate([y1, y2], axis=-1).reshape(_QKV_TILE, DIM)

    q = rot(qkv[:, :DIM])
    k = rot(qkv[:, DIM:2 * DIM])
    out_ref[...] = jnp.concatenate([q, k, qkv[:, 2 * DIM:]], axis=-1)


def _qkv_call(x_flat, w_t, norm_w, cos_t, sin_t):
    grid = (N // _QKV_TILE,)
    n_pos = S // _QKV_TILE
    return pl.pallas_call(
        _qkv_kernel,
        grid=grid,
        in_specs=[
            pl.BlockSpec((_QKV_TILE, DIM), lambda i: (i, 0)),
            pl.BlockSpec((DIM, 3 * DIM), lambda i: (0, 0)),
            pl.BlockSpec((1, DIM), lambda i: (0, 0)),
            pl.BlockSpec((_QKV_TILE, HDIM // 2), lambda i: (i % n_pos, 0)),
            pl.BlockSpec((_QKV_TILE, HDIM // 2), lambda i: (i % n_pos, 0)),
        ],
        out_specs=pl.BlockSpec((_QKV_TILE, 3 * DIM), lambda i: (i, 0)),
        out_shape=jax.ShapeDtypeStruct((N, 3 * DIM), jnp.float32),
    )(x_flat, w_t, norm_w, cos_t, sin_t)


_Q_TILE = 512


def _attn_kernel(q_ref, k_ref, v_ref, out_ref):
    i = pl.program_id(1)
    q = q_ref[0]
    k = k_ref[0]
    v = v_ref[0]
    scale = 1.0 / math.sqrt(HDIM)
    logits = jax.lax.dot_general(
        q, k, (((1,), (1,)), ((), ())),
        preferred_element_type=jnp.float32) * scale
    rows = jax.lax.broadcasted_iota(jnp.int32, (_Q_TILE, L), 0) + i * _Q_TILE
    cols = jax.lax.broadcasted_iota(jnp.int32, (_Q_TILE, L), 1)
    logits = jnp.where(cols <= rows, logits, jnp.float32(-1e30))
    m = jnp.max(logits, axis=-1, keepdims=True)
    p = jnp.exp(logits - m)
    s = jnp.sum(p, axis=-1, keepdims=True)
    o = jnp.dot(p, v, preferred_element_type=jnp.float32)
    out_ref[0] = o / s


def _attn_call(q, k, v):
    grid = (HEADS, L // _Q_TILE)
    return pl.pallas_call(
        _attn_kernel,
        grid=grid,
        in_specs=[
            pl.BlockSpec((1, _Q_TILE, HDIM), lambda h, i: (h, i, 0)),
            pl.BlockSpec((1, L, HDIM), lambda h, i: (h, 0, 0)),
            pl.BlockSpec((1, L, HDIM), lambda h, i: (h, 0, 0)),
        ],
        out_specs=pl.BlockSpec((1, _Q_TILE, HDIM), lambda h, i: (h, i, 0)),
        out_shape=jax.ShapeDtypeStruct((HEADS, L, HDIM), jnp.float32),
    )(q, k, v)


_PROJ_TILE = 512


def _proj_kernel(o_ref, w_ref, x_ref, nw_ref, resid_ref, xffn_ref):
    o = o_ref[...]
    y = jnp.dot(o, w_ref[...], preferred_element_type=jnp.float32)
    resid = y + x_ref[...]
    resid_ref[...] = resid
    xn = resid * jax.lax.rsqrt(
        jnp.mean(resid * resid, axis=-1, keepdims=True) + EPS)
    xffn_ref[...] = xn * nw_ref[...]


def _proj_call(o_flat, w_t, x_flat, norm_w):
    grid = (N // _PROJ_TILE,)
    return pl.pallas_call(
        _proj_kernel,
        grid=grid,
        in_specs=[
            pl.BlockSpec((_PROJ_TILE, DIM), lambda i: (i, 0)),
            pl.BlockSpec((DIM, DIM), lambda i: (0, 0)),
            pl.BlockSpec((_PROJ_TILE, DIM), lambda i: (i, 0)),
            pl.BlockSpec((1, DIM), lambda i: (0, 0)),
        ],
        out_specs=[
            pl.BlockSpec((_PROJ_TILE, DIM), lambda i: (i, 0)),
            pl.BlockSpec((_PROJ_TILE, DIM), lambda i: (i, 0)),
        ],
        out_shape=[
            jax.ShapeDtypeStruct((N, DIM), jnp.float32),
            jax.ShapeDtypeStruct((N, DIM), jnp.float32),
        ],
    )(o_flat, w_t, x_flat, norm_w)


_FFN_TILE = 512


def _ffn_kernel(x_ref, up_ref, down_ref, resid_ref, out_ref):
    x = x_ref[...]
    h = jnp.dot(x, up_ref[...], preferred_element_type=jnp.float32)
    x1 = h[:, :DIM_S]
    x2 = h[:, DIM_S:]
    g = (x1 * jax.lax.logistic(x1)) * x2
    y = jnp.dot(g, down_ref[...], preferred_element_type=jnp.float32)
    out_ref[...] = y + resid_ref[...]


def _ffn_call(x_ffn, up_t, down_t, resid):
    grid = (N // _FFN_TILE,)
    return pl.pallas_call(
        _ffn_kernel,
        grid=grid,
        in_specs=[
            pl.BlockSpec((_FFN_TILE, DIM), lambda i: (i, 0)),
            pl.BlockSpec((DIM, 2 * DIM_S), lambda i: (0, 0)),
            pl.BlockSpec((DIM_S, DIM), lambda i: (0, 0)),
            pl.BlockSpec((_FFN_TILE, DIM), lambda i: (i, 0)),
        ],
        out_specs=pl.BlockSpec((_FFN_TILE, DIM), lambda i: (i, 0)),
        out_shape=jax.ShapeDtypeStruct((N, DIM), jnp.float32),
    )(x_ffn, up_t, down_t, resid)


_RTR_TILE = 512


def _router_kernel(x_ref, keys_ref, idx_ref, vals_ref, bias_ref, comb_ref):
    tok = jnp.dot(x_ref[...], keys_ref[...],
                  preferred_element_type=jnp.float32)
    idx = idx_ref[0]
    onehot = (idx[:, :, None] ==
              jnp.arange(E, dtype=idx.dtype)[None, None, :])
    onehot = onehot.astype(jnp.float32)
    gathered = jnp.sum(onehot * tok[:, None, :], axis=-1)
    gbias = jnp.sum(onehot * bias_ref[...][None, :, :], axis=-1)
    v = vals_ref[0] + gathered + gbias
    sc = jax.lax.logistic(v)
    sc = sc / jnp.sum(sc, axis=-1, keepdims=True)
    comb_ref[0] = jnp.sum(onehot * sc[:, :, None], axis=1).T


def _router_call(x, keys, idx3, vals3, bias):
    grid = (S // _RTR_TILE,)
    return pl.pallas_call(
        _router_kernel,
        grid=grid,
        in_specs=[
            pl.BlockSpec((_RTR_TILE, DIM), lambda i: (i, 0)),
            pl.BlockSpec((DIM, E), lambda i: (0, 0)),
            pl.BlockSpec((1, _RTR_TILE, TOPK), lambda i: (i, 0, 0)),
            pl.BlockSpec((1, _RTR_TILE, TOPK), lambda i: (i, 0, 0)),
            pl.BlockSpec((1, E), lambda i: (0, 0)),
        ],
        out_specs=pl.BlockSpec((1, E, _RTR_TILE), lambda i: (i, 0, 0)),
        out_shape=jax.ShapeDtypeStruct((S // _RTR_TILE, E, _RTR_TILE),
                                       jnp.float32),
    )(x, keys, idx3, vals3, bias)


_MOE_TILE = 512


def _moe_kernel(x_ref, w1_ref, w3_ref, w2_ref, comb_ref, base_ref, out_ref):
    e = pl.program_id(1)

    @pl.when(e == 0)
    def _():
        out_ref[...] = base_ref[...]

    x = x_ref[...]
    h1 = jnp.dot(x, w1_ref[0], preferred_element_type=jnp.float32)
    h3 = jnp.dot(x, w3_ref[0], preferred_element_type=jnp.float32)
    h = (h1 * jax.lax.logistic(h1)) * h3
    y = jax.lax.dot_general(h, w2_ref[0], (((1,), (1,)), ((), ())),
                            preferred_element_type=jnp.float32)
    w = comb_ref[0, 0, :][:, None]
    out_ref[...] += y * w


def _moe_call(x, experts, comb, base):
    grid = (S // _MOE_TILE, E)
    w1, w3, w2 = experts[0], experts[1], experts[2]
    return pl.pallas_call(
        _moe_kernel,
        grid=grid,
        in_specs=[
            pl.BlockSpec((_MOE_TILE, DIM), lambda t, e: (t, 0)),
            pl.BlockSpec((1, DIM, EXP_DIM), lambda t, e: (e, 0, 0)),
            pl.BlockSpec((1, DIM, EXP_DIM), lambda t, e: (e, 0, 0)),
            pl.BlockSpec((1, DIM, EXP_DIM), lambda t, e: (e, 0, 0)),
            pl.BlockSpec((1, 1, _MOE_TILE), lambda t, e: (t * E + e, 0, 0)),
            pl.BlockSpec((_MOE_TILE, DIM), lambda t, e: (t, 0)),
        ],
        out_specs=pl.BlockSpec((_MOE_TILE, DIM), lambda t, e: (t, 0)),
        out_shape=jax.ShapeDtypeStruct((S, DIM), jnp.float32),
    )(x, w1, w3, w2, comb.reshape(-1, 1, _MOE_TILE), base)


def kernel(x_input, p_indices, p_values, f_indices, f_values, attn_w,
           attn_o_w, attn_norm_w, ffn_norm_w, ffn_up_w, ffn_down_w,
           p_ffn_experts, f_ffn_experts, p_token_keys, f_token_keys,
           p_token_router_bias, f_token_router_bias):
    x_flat = x_input.reshape(N, DIM)

    inv_freq = (1.0 / THETA) ** (
        jnp.arange(0, HDIM, 2, dtype=jnp.float32) / HDIM)
    t = jnp.arange(S, dtype=jnp.float32)
    freqs = jnp.outer(t, inv_freq)
    cos_t = jnp.cos(freqs)
    sin_t = jnp.sin(freqs)

    qkv = _qkv_call(x_flat, attn_w.T, attn_norm_w.reshape(1, DIM),
                    cos_t, sin_t)

    def to_heads_interleaved(m):
        return (m.reshape(B, S, HEADS, HDIM)
                 .transpose(2, 1, 0, 3)
                 .reshape(HEADS, L, HDIM))

    q = to_heads_interleaved(qkv[:, :DIM])
    k = to_heads_interleaved(qkv[:, DIM:2 * DIM])
    v = to_heads_interleaved(qkv[:, 2 * DIM:])

    o = _attn_call(q, k, v)

    o_flat = (o.reshape(HEADS, S, B, HDIM)
               .transpose(2, 1, 0, 3)
               .reshape(N, DIM))

    resid, x_ffn = _proj_call(o_flat, attn_o_w.T, x_flat,
                              ffn_norm_w.reshape(1, DIM))

    base = _ffn_call(x_ffn, ffn_up_w.T, ffn_down_w.T, resid)

    p_x = x_ffn[:S]
    f_x = x_ffn[S:]

    def side(x_side, idx, vals, keys, bias, experts, base_side):
        idx3 = idx.reshape(S // _RTR_TILE, _RTR_TILE, TOPK)
        vals3 = vals.reshape(S // _RTR_TILE, _RTR_TILE, TOPK)
        comb = _router_call(x_side, keys, idx3, vals3, bias.reshape(1, E))
        return _moe_call(x_side, experts, comb, base_side)

    py = side(p_x, p_indices, p_values, p_token_keys,
              p_token_router_bias, p_ffn_experts, base[:S])
    fy = side(f_x, f_indices, f_values, f_token_keys,
              f_token_router_bias, f_ffn_experts, base[S:])

    return jnp.concatenate([py, fy], axis=0).reshape(B, S, DIM)


# speedup vs baseline: 3.1947x; 1.4377x over previous
import functools
import math

import jax
import jax.numpy as jnp
from jax.experimental import pallas as pl

DIM = 768
HEADS = 12
HDIM = 64
E = 8
TOPK = 2
EXP_DIM = 256
DIM_S = 2048
EPS = 1e-5
THETA = 10000.0
B = 2
S = 2048
N = B * S
L = 2 * S


_QKV_TILE = 256


def _qkv_kernel(x_ref, w_ref, nw_ref, cos_ref, sin_ref, out_ref):
    x = x_ref[...]
    xn = x * jax.lax.rsqrt(jnp.mean(x * x, axis=-1, keepdims=True) + EPS)
    xn = xn * nw_ref[...]
    qkv = jnp.dot(xn, w_ref[...], preferred_element_type=jnp.float32)
    cos = cos_ref[...][:, None, :]
    sin = sin_ref[...][:, None, :]

    def rot(v):
        v = v.reshape(_QKV_TILE, HEADS, HDIM)
        x1 = v[..., : HDIM // 2]
        x2 = v[..., HDIM // 2:]
        y1 = x1 * cos + x2 * sin
        y2 = -x1 * sin + x2 * cos
        return jnp.concatenate([y1, y2], axis=-1).reshape(_QKV_TILE, DIM)

    q = rot(qkv[:, :DIM])
    k = rot(qkv[:, DIM:2 * DIM])
    out_ref[...] = jnp.concatenate([q, k, qkv[:, 2 * DIM:]], axis=-1)


def _qkv_call(x_flat, w_t, norm_w, cos_t, sin_t):
    grid = (N // _QKV_TILE,)
    n_pos = S // _QKV_TILE
    return pl.pallas_call(
        _qkv_kernel,
        grid=grid,
        in_specs=[
            pl.BlockSpec((_QKV_TILE, DIM), lambda i: (i, 0)),
            pl.BlockSpec((DIM, 3 * DIM), lambda i: (0, 0)),
            pl.BlockSpec((1, DIM), lambda i: (0, 0)),
            pl.BlockSpec((_QKV_TILE, HDIM // 2), lambda i: (i % n_pos, 0)),
            pl.BlockSpec((_QKV_TILE, HDIM // 2), lambda i: (i % n_pos, 0)),
        ],
        out_specs=pl.BlockSpec((_QKV_TILE, 3 * DIM), lambda i: (i, 0)),
        out_shape=jax.ShapeDtypeStruct((N, 3 * DIM), jnp.float32),
    )(x_flat, w_t, norm_w, cos_t, sin_t)


_Q_TILE = 512
_K_TILE = 512


def _attn_kernel(q_ref, k_ref, v_ref, out_ref):
    i = pl.program_id(1)
    c = i // (S // _Q_TILE)
    ib = i % (S // _Q_TILE)
    nb = ib + 1
    scale = 1.0 / math.sqrt(HDIM)
    s_row = (jax.lax.broadcasted_iota(jnp.int32, (_Q_TILE, _K_TILE), 0)
             + ib * _Q_TILE)
    lim0 = s_row
    lim1 = s_row - 1 + c
    cols = jax.lax.broadcasted_iota(jnp.int32, (_Q_TILE, _K_TILE), 1)

    def attend(off):
        q = q_ref[:, off:off + HDIM]

        def body(j, carry):
            m, l, acc = carry
            base = j * _K_TILE
            k0 = k_ref[pl.ds(base, _K_TILE), off:off + HDIM]
            k1 = k_ref[pl.ds(S + base, _K_TILE), off:off + HDIM]
            v0 = v_ref[pl.ds(base, _K_TILE), off:off + HDIM]
            v1 = v_ref[pl.ds(S + base, _K_TILE), off:off + HDIM]
            t = cols + base
            l0 = jax.lax.dot_general(
                q, k0, (((1,), (1,)), ((), ())),
                preferred_element_type=jnp.float32) * scale
            l1 = jax.lax.dot_general(
                q, k1, (((1,), (1,)), ((), ())),
                preferred_element_type=jnp.float32) * scale
            l0 = jnp.where(t <= lim0, l0, jnp.float32(-1e30))
            l1 = jnp.where(t <= lim1, l1, jnp.float32(-1e30))
            m_new = jnp.maximum(m, jnp.maximum(
                jnp.max(l0, axis=-1, keepdims=True),
                jnp.max(l1, axis=-1, keepdims=True)))
            p0 = jnp.exp(l0 - m_new)
            p1 = jnp.exp(l1 - m_new)
            alpha = jnp.exp(m - m_new)
            l_new = l * alpha + jnp.sum(p0, axis=-1, keepdims=True) \
                + jnp.sum(p1, axis=-1, keepdims=True)
            acc_new = acc * alpha \
                + jnp.dot(p0, v0, preferred_element_type=jnp.float32) \
                + jnp.dot(p1, v1, preferred_element_type=jnp.float32)
            return m_new, l_new, acc_new

        m0 = jnp.full((_Q_TILE, 1), -1e30, dtype=jnp.float32)
        l0_ = jnp.zeros((_Q_TILE, 1), dtype=jnp.float32)
        a0 = jnp.zeros((_Q_TILE, HDIM), dtype=jnp.float32)
        m, l, acc = jax.lax.fori_loop(0, nb, body, (m0, l0_, a0))
        return acc / l

    out_ref[...] = jnp.concatenate([attend(0), attend(HDIM)], axis=1)


def _attn_call(qkv):
    grid = (HEADS // 2, N // _Q_TILE)
    return pl.pallas_call(
        _attn_kernel,
        grid=grid,
        in_specs=[
            pl.BlockSpec((_Q_TILE, 2 * HDIM), lambda g, i: (i, g)),
            pl.BlockSpec((N, 2 * HDIM), lambda g, i: (0, HEADS // 2 + g)),
            pl.BlockSpec((N, 2 * HDIM), lambda g, i: (0, HEADS + g)),
        ],
        out_specs=pl.BlockSpec((_Q_TILE, 2 * HDIM), lambda g, i: (i, g)),
        out_shape=jax.ShapeDtypeStruct((N, DIM), jnp.float32),
    )(qkv, qkv, qkv)


_PROJ_TILE = 512


def _proj_kernel(o_ref, w_ref, x_ref, nw_ref, resid_ref, xffn_ref):
    o = o_ref[...]
    y = jnp.dot(o, w_ref[...], preferred_element_type=jnp.float32)
    resid = y + x_ref[...]
    resid_ref[...] = resid
    xn = resid * jax.lax.rsqrt(
        jnp.mean(resid * resid, axis=-1, keepdims=True) + EPS)
    xffn_ref[...] = xn * nw_ref[...]


def _proj_call(o_flat, w_t, x_flat, norm_w):
    grid = (N // _PROJ_TILE,)
    return pl.pallas_call(
        _proj_kernel,
        grid=grid,
        in_specs=[
            pl.BlockSpec((_PROJ_TILE, DIM), lambda i: (i, 0)),
            pl.BlockSpec((DIM, DIM), lambda i: (0, 0)),
            pl.BlockSpec((_PROJ_TILE, DIM), lambda i: (i, 0)),
            pl.BlockSpec((1, DIM), lambda i: (0, 0)),
        ],
        out_specs=[
            pl.BlockSpec((_PROJ_TILE, DIM), lambda i: (i, 0)),
            pl.BlockSpec((_PROJ_TILE, DIM), lambda i: (i, 0)),
        ],
        out_shape=[
            jax.ShapeDtypeStruct((N, DIM), jnp.float32),
            jax.ShapeDtypeStruct((N, DIM), jnp.float32),
        ],
    )(o_flat, w_t, x_flat, norm_w)


_FFN_TILE = 512


def _ffn_kernel(x_ref, up_ref, down_ref, resid_ref, out_ref):
    x = x_ref[...]
    h = jnp.dot(x, up_ref[...], preferred_element_type=jnp.float32)
    x1 = h[:, :DIM_S]
    x2 = h[:, DIM_S:]
    g = (x1 * jax.lax.logistic(x1)) * x2
    y = jnp.dot(g, down_ref[...], preferred_element_type=jnp.float32)
    out_ref[...] = y + resid_ref[...]


def _ffn_call(x_ffn, up_t, down_t, resid):
    grid = (N // _FFN_TILE,)
    return pl.pallas_call(
        _ffn_kernel,
        grid=grid,
        in_specs=[
            pl.BlockSpec((_FFN_TILE, DIM), lambda i: (i, 0)),
            pl.BlockSpec((DIM, 2 * DIM_S), lambda i: (0, 0)),
            pl.BlockSpec((DIM_S, DIM), lambda i: (0, 0)),
            pl.BlockSpec((_FFN_TILE, DIM), lambda i: (i, 0)),
        ],
        out_specs=pl.BlockSpec((_FFN_TILE, DIM), lambda i: (i, 0)),
        out_shape=jax.ShapeDtypeStruct((N, DIM), jnp.float32),
    )(x_ffn, up_t, down_t, resid)


_RTR_TILE = 512


def _router_kernel(x_ref, keys_ref, idx_ref, vals_ref, bias_ref, comb_ref):
    tok = jnp.dot(x_ref[...], keys_ref[...],
                  preferred_element_type=jnp.float32)
    idx = idx_ref[0]
    onehot = (idx[:, :, None] ==
              jnp.arange(E, dtype=idx.dtype)[None, None, :])
    onehot = onehot.astype(jnp.float32)
    gathered = jnp.sum(onehot * tok[:, None, :], axis=-1)
    gbias = jnp.sum(onehot * bias_ref[...][None, :, :], axis=-1)
    v = vals_ref[0] + gathered + gbias
    sc = jax.lax.logistic(v)
    sc = sc / jnp.sum(sc, axis=-1, keepdims=True)
    comb_ref[0] = jnp.sum(onehot * sc[:, :, None], axis=1).T


def _router_call(x, keys, idx3, vals3, bias):
    grid = (S // _RTR_TILE,)
    return pl.pallas_call(
        _router_kernel,
        grid=grid,
        in_specs=[
            pl.BlockSpec((_RTR_TILE, DIM), lambda i: (i, 0)),
            pl.BlockSpec((DIM, E), lambda i: (0, 0)),
            pl.BlockSpec((1, _RTR_TILE, TOPK), lambda i: (i, 0, 0)),
            pl.BlockSpec((1, _RTR_TILE, TOPK), lambda i: (i, 0, 0)),
            pl.BlockSpec((1, E), lambda i: (0, 0)),
        ],
        out_specs=pl.BlockSpec((1, E, _RTR_TILE), lambda i: (i, 0, 0)),
        out_shape=jax.ShapeDtypeStruct((S // _RTR_TILE, E, _RTR_TILE),
                                       jnp.float32),
    )(x, keys, idx3, vals3, bias)


_MOE_TILE = 512


def _moe_kernel(x_ref, w1_ref, w3_ref, w2_ref, comb_ref, base_ref, out_ref):
    e = pl.program_id(1)

    @pl.when(e == 0)
    def _():
        out_ref[...] = base_ref[...]

    x = x_ref[...]
    h1 = jnp.dot(x, w1_ref[0], preferred_element_type=jnp.float32)
    h3 = jnp.dot(x, w3_ref[0], preferred_element_type=jnp.float32)
    h = (h1 * jax.lax.logistic(h1)) * h3
    y = jax.lax.dot_general(h, w2_ref[0], (((1,), (1,)), ((), ())),
                            preferred_element_type=jnp.float32)
    w = comb_ref[0, 0, :][:, None]
    out_ref[...] += y * w


def _moe_call(x, experts, comb, base):
    grid = (S // _MOE_TILE, E)
    w1, w3, w2 = experts[0], experts[1], experts[2]
    return pl.pallas_call(
        _moe_kernel,
        grid=grid,
        in_specs=[
            pl.BlockSpec((_MOE_TILE, DIM), lambda t, e: (t, 0)),
            pl.BlockSpec((1, DIM, EXP_DIM), lambda t, e: (e, 0, 0)),
            pl.BlockSpec((1, DIM, EXP_DIM), lambda t, e: (e, 0, 0)),
            pl.BlockSpec((1, DIM, EXP_DIM), lambda t, e: (e, 0, 0)),
            pl.BlockSpec((1, 1, _MOE_TILE), lambda t, e: (t * E + e, 0, 0)),
            pl.BlockSpec((_MOE_TILE, DIM), lambda t, e: (t, 0)),
        ],
        out_specs=pl.BlockSpec((_MOE_TILE, DIM), lambda t, e: (t, 0)),
        out_shape=jax.ShapeDtypeStruct((S, DIM), jnp.float32),
    )(x, w1, w3, w2, comb.reshape(-1, 1, _MOE_TILE), base)


def kernel(x_input, p_indices, p_values, f_indices, f_values, attn_w,
           attn_o_w, attn_norm_w, ffn_norm_w, ffn_up_w, ffn_down_w,
           p_ffn_experts, f_ffn_experts, p_token_keys, f_token_keys,
           p_token_router_bias, f_token_router_bias):
    x_flat = x_input.reshape(N, DIM)

    inv_freq = (1.0 / THETA) ** (
        jnp.arange(0, HDIM, 2, dtype=jnp.float32) / HDIM)
    t = jnp.arange(S, dtype=jnp.float32)
    freqs = jnp.outer(t, inv_freq)
    cos_t = jnp.cos(freqs)
    sin_t = jnp.sin(freqs)

    qkv = _qkv_call(x_flat, attn_w.T, attn_norm_w.reshape(1, DIM),
                    cos_t, sin_t)

    o_flat = _attn_call(qkv)

    resid, x_ffn = _proj_call(o_flat, attn_o_w.T, x_flat,
                              ffn_norm_w.reshape(1, DIM))

    base = _ffn_call(x_ffn, ffn_up_w.T, ffn_down_w.T, resid)

    p_x = x_ffn[:S]
    f_x = x_ffn[S:]

    def side(x_side, idx, vals, keys, bias, experts, base_side):
        idx3 = idx.reshape(S // _RTR_TILE, _RTR_TILE, TOPK)
        vals3 = vals.reshape(S // _RTR_TILE, _RTR_TILE, TOPK)
        comb = _router_call(x_side, keys, idx3, vals3, bias.reshape(1, E))
        return _moe_call(x_side, experts, comb, base_side)

    py = side(p_x, p_indices, p_values, p_token_keys,
              p_token_router_bias, p_ffn_experts, base[:S])
    fy = side(f_x, f_indices, f_values, f_token_keys,
              f_token_router_bias, f_ffn_experts, base[S:])

    return jnp.concatenate([py, fy], axis=0).reshape(B, S, DIM)


# speedup vs baseline: 3.3295x; 1.0422x over previous
import functools
import math

import jax
import jax.numpy as jnp
from jax.experimental import pallas as pl

DIM = 768
HEADS = 12
HDIM = 64
E = 8
TOPK = 2
EXP_DIM = 256
DIM_S = 2048
EPS = 1e-5
THETA = 10000.0
B = 2
S = 2048
N = B * S
L = 2 * S


_QKV_TILE = 256


def _qkv_kernel(x_ref, w_ref, nw_ref, cos_ref, sin_ref, out_ref):
    x = x_ref[...]
    xn = x * jax.lax.rsqrt(jnp.mean(x * x, axis=-1, keepdims=True) + EPS)
    xn = (xn * nw_ref[...]).astype(jnp.bfloat16)
    qkv = jnp.dot(xn, w_ref[...], preferred_element_type=jnp.float32)
    cos = cos_ref[...][:, None, :]
    sin = sin_ref[...][:, None, :]

    def rot(v):
        v = v.reshape(_QKV_TILE, HEADS, HDIM)
        x1 = v[..., : HDIM // 2]
        x2 = v[..., HDIM // 2:]
        y1 = x1 * cos + x2 * sin
        y2 = -x1 * sin + x2 * cos
        return jnp.concatenate([y1, y2], axis=-1).reshape(_QKV_TILE, DIM)

    q = rot(qkv[:, :DIM])
    k = rot(qkv[:, DIM:2 * DIM])
    out_ref[...] = jnp.concatenate(
        [q, k, qkv[:, 2 * DIM:]], axis=-1).astype(jnp.bfloat16)


def _qkv_call(x_flat, w_t, norm_w, cos_t, sin_t):
    grid = (N // _QKV_TILE,)
    n_pos = S // _QKV_TILE
    return pl.pallas_call(
        _qkv_kernel,
        grid=grid,
        in_specs=[
            pl.BlockSpec((_QKV_TILE, DIM), lambda i: (i, 0)),
            pl.BlockSpec((DIM, 3 * DIM), lambda i: (0, 0)),
            pl.BlockSpec((1, DIM), lambda i: (0, 0)),
            pl.BlockSpec((_QKV_TILE, HDIM // 2), lambda i: (i % n_pos, 0)),
            pl.BlockSpec((_QKV_TILE, HDIM // 2), lambda i: (i % n_pos, 0)),
        ],
        out_specs=pl.BlockSpec((_QKV_TILE, 3 * DIM), lambda i: (i, 0)),
        out_shape=jax.ShapeDtypeStruct((N, 3 * DIM), jnp.bfloat16),
    )(x_flat, w_t, norm_w, cos_t, sin_t)


_Q_TILE = 512
_K_TILE = 512


def _attn_kernel(q_ref, k_ref, v_ref, out_ref):
    i = pl.program_id(1)
    c = i // (S // _Q_TILE)
    ib = i % (S // _Q_TILE)
    nb = ib + 1
    scale = 1.0 / math.sqrt(HDIM)
    s_row = (jax.lax.broadcasted_iota(jnp.int32, (_Q_TILE, _K_TILE), 0)
             + ib * _Q_TILE)
    lim0 = s_row
    lim1 = s_row - 1 + c
    cols = jax.lax.broadcasted_iota(jnp.int32, (_Q_TILE, _K_TILE), 1)

    def attend(off):
        q = q_ref[:, off:off + HDIM]

        def body(j, carry):
            m, l, acc = carry
            base = j * _K_TILE
            k0 = k_ref[pl.ds(base, _K_TILE), off:off + HDIM]
            k1 = k_ref[pl.ds(S + base, _K_TILE), off:off + HDIM]
            v0 = v_ref[pl.ds(base, _K_TILE), off:off + HDIM]
            v1 = v_ref[pl.ds(S + base, _K_TILE), off:off + HDIM]
            t = cols + base
            l0 = jax.lax.dot_general(
                q, k0, (((1,), (1,)), ((), ())),
                preferred_element_type=jnp.float32) * scale
            l1 = jax.lax.dot_general(
                q, k1, (((1,), (1,)), ((), ())),
                preferred_element_type=jnp.float32) * scale
            l0 = jnp.where(t <= lim0, l0, jnp.float32(-1e30))
            l1 = jnp.where(t <= lim1, l1, jnp.float32(-1e30))
            m_new = jnp.maximum(m, jnp.maximum(
                jnp.max(l0, axis=-1, keepdims=True),
                jnp.max(l1, axis=-1, keepdims=True)))
            p0 = jnp.exp(l0 - m_new)
            p1 = jnp.exp(l1 - m_new)
            alpha = jnp.exp(m - m_new)
            l_new = l * alpha + jnp.sum(p0, axis=-1, keepdims=True) \
                + jnp.sum(p1, axis=-1, keepdims=True)
            acc_new = acc * alpha \
                + jnp.dot(p0.astype(jnp.bfloat16), v0,
                          preferred_element_type=jnp.float32) \
                + jnp.dot(p1.astype(jnp.bfloat16), v1,
                          preferred_element_type=jnp.float32)
            return m_new, l_new, acc_new

        m0 = jnp.full((_Q_TILE, 1), -1e30, dtype=jnp.float32)
        l0_ = jnp.zeros((_Q_TILE, 1), dtype=jnp.float32)
        a0 = jnp.zeros((_Q_TILE, HDIM), dtype=jnp.float32)
        m, l, acc = jax.lax.fori_loop(0, nb, body, (m0, l0_, a0))
        return acc / l

    out_ref[...] = jnp.concatenate(
        [attend(0), attend(HDIM)], axis=1).astype(jnp.bfloat16)


def _attn_call(qkv):
    grid = (HEADS // 2, N // _Q_TILE)
    return pl.pallas_call(
        _attn_kernel,
        grid=grid,
        in_specs=[
            pl.BlockSpec((_Q_TILE, 2 * HDIM), lambda g, i: (i, g)),
            pl.BlockSpec((N, 2 * HDIM), lambda g, i: (0, HEADS // 2 + g)),
            pl.BlockSpec((N, 2 * HDIM), lambda g, i: (0, HEADS + g)),
        ],
        out_specs=pl.BlockSpec((_Q_TILE, 2 * HDIM), lambda g, i: (i, g)),
        out_shape=jax.ShapeDtypeStruct((N, DIM), jnp.bfloat16),
    )(qkv, qkv, qkv)


_PROJ_TILE = 512


def _proj_kernel(o_ref, w_ref, x_ref, nw_ref, resid_ref, xffn_ref):
    o = o_ref[...]
    y = jnp.dot(o, w_ref[...], preferred_element_type=jnp.float32)
    resid = y + x_ref[...]
    resid_ref[...] = resid
    xn = resid * jax.lax.rsqrt(
        jnp.mean(resid * resid, axis=-1, keepdims=True) + EPS)
    xffn_ref[...] = (xn * nw_ref[...]).astype(jnp.bfloat16)


def _proj_call(o_flat, w_t, x_flat, norm_w):
    grid = (N // _PROJ_TILE,)
    return pl.pallas_call(
        _proj_kernel,
        grid=grid,
        in_specs=[
            pl.BlockSpec((_PROJ_TILE, DIM), lambda i: (i, 0)),
            pl.BlockSpec((DIM, DIM), lambda i: (0, 0)),
            pl.BlockSpec((_PROJ_TILE, DIM), lambda i: (i, 0)),
            pl.BlockSpec((1, DIM), lambda i: (0, 0)),
        ],
        out_specs=[
            pl.BlockSpec((_PROJ_TILE, DIM), lambda i: (i, 0)),
            pl.BlockSpec((_PROJ_TILE, DIM), lambda i: (i, 0)),
        ],
        out_shape=[
            jax.ShapeDtypeStruct((N, DIM), jnp.float32),
            jax.ShapeDtypeStruct((N, DIM), jnp.bfloat16),
        ],
    )(o_flat, w_t, x_flat, norm_w)


_FFN_TILE = 512


def _ffn_kernel(x_ref, up_ref, down_ref, resid_ref, out_ref):
    x = x_ref[...]
    h = jnp.dot(x, up_ref[...], preferred_element_type=jnp.float32)
    x1 = h[:, :DIM_S]
    x2 = h[:, DIM_S:]
    g = ((x1 * jax.lax.logistic(x1)) * x2).astype(jnp.bfloat16)
    y = jnp.dot(g, down_ref[...], preferred_element_type=jnp.float32)
    out_ref[...] = y + resid_ref[...]


def _ffn_call(x_ffn, up_t, down_t, resid):
    grid = (N // _FFN_TILE,)
    return pl.pallas_call(
        _ffn_kernel,
        grid=grid,
        in_specs=[
            pl.BlockSpec((_FFN_TILE, DIM), lambda i: (i, 0)),
            pl.BlockSpec((DIM, 2 * DIM_S), lambda i: (0, 0)),
            pl.BlockSpec((DIM_S, DIM), lambda i: (0, 0)),
            pl.BlockSpec((_FFN_TILE, DIM), lambda i: (i, 0)),
        ],
        out_specs=pl.BlockSpec((_FFN_TILE, DIM), lambda i: (i, 0)),
        out_shape=jax.ShapeDtypeStruct((N, DIM), jnp.float32),
    )(x_ffn, up_t, down_t, resid)


_RTR_TILE = 512


def _router_kernel(x_ref, keys_ref, idx_ref, vals_ref, bias_ref, comb_ref):
    tok = jnp.dot(x_ref[...], keys_ref[...],
                  preferred_element_type=jnp.float32)
    idx = idx_ref[0]
    onehot = (idx[:, :, None] ==
              jnp.arange(E, dtype=idx.dtype)[None, None, :])
    onehot = onehot.astype(jnp.float32)
    gathered = jnp.sum(onehot * tok[:, None, :], axis=-1)
    gbias = jnp.sum(onehot * bias_ref[...][None, :, :], axis=-1)
    v = vals_ref[0] + gathered + gbias
    sc = jax.lax.logistic(v)
    sc = sc / jnp.sum(sc, axis=-1, keepdims=True)
    comb_ref[0] = jnp.sum(onehot * sc[:, :, None], axis=1).T


def _router_call(x, keys, idx3, vals3, bias):
    grid = (S // _RTR_TILE,)
    return pl.pallas_call(
        _router_kernel,
        grid=grid,
        in_specs=[
            pl.BlockSpec((_RTR_TILE, DIM), lambda i: (i, 0)),
            pl.BlockSpec((DIM, E), lambda i: (0, 0)),
            pl.BlockSpec((1, _RTR_TILE, TOPK), lambda i: (i, 0, 0)),
            pl.BlockSpec((1, _RTR_TILE, TOPK), lambda i: (i, 0, 0)),
            pl.BlockSpec((1, E), lambda i: (0, 0)),
        ],
        out_specs=pl.BlockSpec((1, E, _RTR_TILE), lambda i: (i, 0, 0)),
        out_shape=jax.ShapeDtypeStruct((S // _RTR_TILE, E, _RTR_TILE),
                                       jnp.float32),
    )(x, keys, idx3, vals3, bias)


_MOE_TILE = 512


def _moe_kernel(x_ref, w1_ref, w3_ref, w2_ref, comb_ref, base_ref, out_ref):
    e = pl.program_id(1)

    @pl.when(e == 0)
    def _():
        out_ref[...] = base_ref[...]

    x = x_ref[...]
    h1 = jnp.dot(x, w1_ref[0], preferred_element_type=jnp.float32)
    h3 = jnp.dot(x, w3_ref[0], preferred_element_type=jnp.float32)
    h = ((h1 * jax.lax.logistic(h1)) * h3).astype(jnp.bfloat16)
    y = jax.lax.dot_general(h, w2_ref[0], (((1,), (1,)), ((), ())),
                            preferred_element_type=jnp.float32)
    w = comb_ref[0, 0, :][:, None]
    out_ref[...] += y * w


def _moe_call(x, experts, comb, base):
    grid = (S // _MOE_TILE, E)
    w1, w3, w2 = experts[0], experts[1], experts[2]
    return pl.pallas_call(
        _moe_kernel,
        grid=grid,
        in_specs=[
            pl.BlockSpec((_MOE_TILE, DIM), lambda t, e: (t, 0)),
            pl.BlockSpec((1, DIM, EXP_DIM), lambda t, e: (e, 0, 0)),
            pl.BlockSpec((1, DIM, EXP_DIM), lambda t, e: (e, 0, 0)),
            pl.BlockSpec((1, DIM, EXP_DIM), lambda t, e: (e, 0, 0)),
            pl.BlockSpec((1, 1, _MOE_TILE), lambda t, e: (t * E + e, 0, 0)),
            pl.BlockSpec((_MOE_TILE, DIM), lambda t, e: (t, 0)),
        ],
        out_specs=pl.BlockSpec((_MOE_TILE, DIM), lambda t, e: (t, 0)),
        out_shape=jax.ShapeDtypeStruct((S, DIM), jnp.float32),
    )(x, w1, w3, w2, comb.reshape(-1, 1, _MOE_TILE), base)


def kernel(x_input, p_indices, p_values, f_indices, f_values, attn_w,
           attn_o_w, attn_norm_w, ffn_norm_w, ffn_up_w, ffn_down_w,
           p_ffn_experts, f_ffn_experts, p_token_keys, f_token_keys,
           p_token_router_bias, f_token_router_bias):
    x_flat = x_input.reshape(N, DIM)

    inv_freq = (1.0 / THETA) ** (
        jnp.arange(0, HDIM, 2, dtype=jnp.float32) / HDIM)
    t = jnp.arange(S, dtype=jnp.float32)
    freqs = jnp.outer(t, inv_freq)
    cos_t = jnp.cos(freqs)
    sin_t = jnp.sin(freqs)

    qkv = _qkv_call(x_flat, attn_w.T.astype(jnp.bfloat16),
                    attn_norm_w.reshape(1, DIM), cos_t, sin_t)

    o_flat = _attn_call(qkv)

    resid, x_ffn = _proj_call(o_flat, attn_o_w.T.astype(jnp.bfloat16),
                              x_flat, ffn_norm_w.reshape(1, DIM))

    base = _ffn_call(x_ffn, ffn_up_w.T.astype(jnp.bfloat16),
                     ffn_down_w.T.astype(jnp.bfloat16), resid)

    p_x = x_ffn[:S]
    f_x = x_ffn[S:]

    def side(x_side, idx, vals, keys, bias, experts, base_side):
        idx3 = idx.reshape(S // _RTR_TILE, _RTR_TILE, TOPK)
        vals3 = vals.reshape(S // _RTR_TILE, _RTR_TILE, TOPK)
        comb = _router_call(x_side, keys.astype(jnp.bfloat16), idx3, vals3,
                            bias.reshape(1, E))
        return _moe_call(x_side, experts.astype(jnp.bfloat16), comb,
                         base_side)

    py = side(p_x, p_indices, p_values, p_token_keys,
              p_token_router_bias, p_ffn_experts, base[:S])
    fy = side(f_x, f_indices, f_values, f_token_keys,
              f_token_router_bias, f_ffn_experts, base[S:])

    return jnp.concatenate([py, fy], axis=0).reshape(B, S, DIM)


# speedup vs baseline: 3.3640x; 1.0103x over previous
import functools
import math

import jax
import jax.numpy as jnp
from jax.experimental import pallas as pl

DIM = 768
HEADS = 12
HDIM = 64
E = 8
TOPK = 2
EXP_DIM = 256
DIM_S = 2048
EPS = 1e-5
THETA = 10000.0
B = 2
S = 2048
N = B * S
L = 2 * S


_QKV_TILE = 256


def _qkv_kernel(x_ref, w_ref, nw_ref, cos_ref, sin_ref, out_ref):
    x = x_ref[...]
    xn = x * jax.lax.rsqrt(jnp.mean(x * x, axis=-1, keepdims=True) + EPS)
    xn = (xn * nw_ref[...]).astype(jnp.bfloat16)
    qkv = jnp.dot(xn, w_ref[...], preferred_element_type=jnp.float32)
    cos = cos_ref[...]
    sin = sin_ref[...]
    q = qkv[:, :DIM] * cos + qkv[:, 3 * DIM:4 * DIM] * sin
    k = qkv[:, DIM:2 * DIM] * cos + qkv[:, 4 * DIM:] * sin
    out_ref[...] = jnp.concatenate(
        [q, k, qkv[:, 2 * DIM:3 * DIM]], axis=-1).astype(jnp.bfloat16)


def _qkv_call(x_flat, w_aug, norm_w, cos_t, sin_t):
    grid = (N // _QKV_TILE,)
    n_pos = S // _QKV_TILE
    return pl.pallas_call(
        _qkv_kernel,
        grid=grid,
        in_specs=[
            pl.BlockSpec((_QKV_TILE, DIM), lambda i: (i, 0)),
            pl.BlockSpec((DIM, 5 * DIM), lambda i: (0, 0)),
            pl.BlockSpec((1, DIM), lambda i: (0, 0)),
            pl.BlockSpec((_QKV_TILE, DIM), lambda i: (i % n_pos, 0)),
            pl.BlockSpec((_QKV_TILE, DIM), lambda i: (i % n_pos, 0)),
        ],
        out_specs=pl.BlockSpec((_QKV_TILE, 3 * DIM), lambda i: (i, 0)),
        out_shape=jax.ShapeDtypeStruct((N, 3 * DIM), jnp.bfloat16),
    )(x_flat, w_aug, norm_w, cos_t, sin_t)


_Q_TILE = 512
_K_TILE = 512


def _attn_kernel(q_ref, k_ref, v_ref, out_ref):
    i = pl.program_id(1)
    c = i // (S // _Q_TILE)
    ib = i % (S // _Q_TILE)
    nb = ib + 1
    scale = 1.0 / math.sqrt(HDIM)
    s_row = (jax.lax.broadcasted_iota(jnp.int32, (_Q_TILE, _K_TILE), 0)
             + ib * _Q_TILE)
    lim0 = s_row
    lim1 = s_row - 1 + c
    cols = jax.lax.broadcasted_iota(jnp.int32, (_Q_TILE, _K_TILE), 1)

    def attend(off):
        q = q_ref[:, off:off + HDIM]

        def body(j, carry):
            m, l, acc = carry
            base = j * _K_TILE
            k0 = k_ref[pl.ds(base, _K_TILE), off:off + HDIM]
            k1 = k_ref[pl.ds(S + base, _K_TILE), off:off + HDIM]
            v0 = v_ref[pl.ds(base, _K_TILE), off:off + HDIM]
            v1 = v_ref[pl.ds(S + base, _K_TILE), off:off + HDIM]
            t = cols + base
            l0 = jax.lax.dot_general(
                q, k0, (((1,), (1,)), ((), ())),
                preferred_element_type=jnp.float32) * scale
            l1 = jax.lax.dot_general(
                q, k1, (((1,), (1,)), ((), ())),
                preferred_element_type=jnp.float32) * scale
            l0 = jnp.where(t <= lim0, l0, jnp.float32(-1e30))
            l1 = jnp.where(t <= lim1, l1, jnp.float32(-1e30))
            m_new = jnp.maximum(m, jnp.maximum(
                jnp.max(l0, axis=-1, keepdims=True),
                jnp.max(l1, axis=-1, keepdims=True)))
            p0 = jnp.exp(l0 - m_new)
            p1 = jnp.exp(l1 - m_new)
            alpha = jnp.exp(m - m_new)
            l_new = l * alpha + jnp.sum(p0, axis=-1, keepdims=True) \
                + jnp.sum(p1, axis=-1, keepdims=True)
            acc_new = acc * alpha \
                + jnp.dot(p0.astype(jnp.bfloat16), v0,
                          preferred_element_type=jnp.float32) \
                + jnp.dot(p1.astype(jnp.bfloat16), v1,
                          preferred_element_type=jnp.float32)
            return m_new, l_new, acc_new

        m0 = jnp.full((_Q_TILE, 1), -1e30, dtype=jnp.float32)
        l0_ = jnp.zeros((_Q_TILE, 1), dtype=jnp.float32)
        a0 = jnp.zeros((_Q_TILE, HDIM), dtype=jnp.float32)
        m, l, acc = jax.lax.fori_loop(0, nb, body, (m0, l0_, a0))
        return acc / l

    out_ref[...] = jnp.concatenate(
        [attend(0), attend(HDIM)], axis=1).astype(jnp.bfloat16)


def _attn_call(qkv):
    grid = (HEADS // 2, N // _Q_TILE)
    return pl.pallas_call(
        _attn_kernel,
        grid=grid,
        in_specs=[
            pl.BlockSpec((_Q_TILE, 2 * HDIM), lambda g, i: (i, g)),
            pl.BlockSpec((N, 2 * HDIM), lambda g, i: (0, HEADS // 2 + g)),
            pl.BlockSpec((N, 2 * HDIM), lambda g, i: (0, HEADS + g)),
        ],
        out_specs=pl.BlockSpec((_Q_TILE, 2 * HDIM), lambda g, i: (i, g)),
        out_shape=jax.ShapeDtypeStruct((N, DIM), jnp.bfloat16),
    )(qkv, qkv, qkv)


_PROJ_TILE = 512


def _proj_kernel(o_ref, w_ref, x_ref, nw_ref, resid_ref, xffn_ref):
    o = o_ref[...]
    y = jnp.dot(o, w_ref[...], preferred_element_type=jnp.float32)
    resid = y + x_ref[...]
    resid_ref[...] = resid
    xn = resid * jax.lax.rsqrt(
        jnp.mean(resid * resid, axis=-1, keepdims=True) + EPS)
    xffn_ref[...] = (xn * nw_ref[...]).astype(jnp.bfloat16)


def _proj_call(o_flat, w_t, x_flat, norm_w):
    grid = (N // _PROJ_TILE,)
    return pl.pallas_call(
        _proj_kernel,
        grid=grid,
        in_specs=[
            pl.BlockSpec((_PROJ_TILE, DIM), lambda i: (i, 0)),
            pl.BlockSpec((DIM, DIM), lambda i: (0, 0)),
            pl.BlockSpec((_PROJ_TILE, DIM), lambda i: (i, 0)),
            pl.BlockSpec((1, DIM), lambda i: (0, 0)),
        ],
        out_specs=[
            pl.BlockSpec((_PROJ_TILE, DIM), lambda i: (i, 0)),
            pl.BlockSpec((_PROJ_TILE, DIM), lambda i: (i, 0)),
        ],
        out_shape=[
            jax.ShapeDtypeStruct((N, DIM), jnp.float32),
            jax.ShapeDtypeStruct((N, DIM), jnp.bfloat16),
        ],
    )(o_flat, w_t, x_flat, norm_w)


_FFN_TILE = 512


def _ffn_kernel(x_ref, up_ref, down_ref, resid_ref, out_ref):
    x = x_ref[...]
    h = jnp.dot(x, up_ref[...], preferred_element_type=jnp.float32)
    x1 = h[:, :DIM_S]
    x2 = h[:, DIM_S:]
    g = ((x1 * jax.lax.logistic(x1)) * x2).astype(jnp.bfloat16)
    y = jnp.dot(g, down_ref[...], preferred_element_type=jnp.float32)
    out_ref[...] = y + resid_ref[...]


def _ffn_call(x_ffn, up_t, down_t, resid):
    grid = (N // _FFN_TILE,)
    return pl.pallas_call(
        _ffn_kernel,
        grid=grid,
        in_specs=[
            pl.BlockSpec((_FFN_TILE, DIM), lambda i: (i, 0)),
            pl.BlockSpec((DIM, 2 * DIM_S), lambda i: (0, 0)),
            pl.BlockSpec((DIM_S, DIM), lambda i: (0, 0)),
            pl.BlockSpec((_FFN_TILE, DIM), lambda i: (i, 0)),
        ],
        out_specs=pl.BlockSpec((_FFN_TILE, DIM), lambda i: (i, 0)),
        out_shape=jax.ShapeDtypeStruct((N, DIM), jnp.float32),
    )(x_ffn, up_t, down_t, resid)


_RTR_TILE = 512


def _router_kernel(x_ref, keys_ref, idx_ref, vals_ref, bias_ref, comb_ref):
    tok = jnp.dot(x_ref[...], keys_ref[...],
                  preferred_element_type=jnp.float32)
    idx = idx_ref[0]
    onehot = (idx[:, :, None] ==
              jnp.arange(E, dtype=idx.dtype)[None, None, :])
    onehot = onehot.astype(jnp.float32)
    gathered = jnp.sum(onehot * tok[:, None, :], axis=-1)
    gbias = jnp.sum(onehot * bias_ref[...][None, :, :], axis=-1)
    v = vals_ref[0] + gathered + gbias
    sc = jax.lax.logistic(v)
    sc = sc / jnp.sum(sc, axis=-1, keepdims=True)
    comb_ref[0] = jnp.sum(onehot * sc[:, :, None], axis=1).T


def _router_call(x, keys, idx3, vals3, bias):
    grid = (S // _RTR_TILE,)
    return pl.pallas_call(
        _router_kernel,
        grid=grid,
        in_specs=[
            pl.BlockSpec((_RTR_TILE, DIM), lambda i: (i, 0)),
            pl.BlockSpec((DIM, E), lambda i: (0, 0)),
            pl.BlockSpec((1, _RTR_TILE, TOPK), lambda i: (i, 0, 0)),
            pl.BlockSpec((1, _RTR_TILE, TOPK), lambda i: (i, 0, 0)),
            pl.BlockSpec((1, E), lambda i: (0, 0)),
        ],
        out_specs=pl.BlockSpec((1, E, _RTR_TILE), lambda i: (i, 0, 0)),
        out_shape=jax.ShapeDtypeStruct((S // _RTR_TILE, E, _RTR_TILE),
                                       jnp.float32),
    )(x, keys, idx3, vals3, bias)


_MOE_TILE = 512


def _moe_kernel(x_ref, w1_ref, w3_ref, w2_ref, comb_ref, base_ref, out_ref):
    e = pl.program_id(1)

    @pl.when(e == 0)
    def _():
        out_ref[...] = base_ref[...]

    x = x_ref[...]
    h1 = jnp.dot(x, w1_ref[0], preferred_element_type=jnp.float32)
    h3 = jnp.dot(x, w3_ref[0], preferred_element_type=jnp.float32)
    h = ((h1 * jax.lax.logistic(h1)) * h3).astype(jnp.bfloat16)
    y = jax.lax.dot_general(h, w2_ref[0], (((1,), (1,)), ((), ())),
                            preferred_element_type=jnp.float32)
    w = comb_ref[0, 0, :][:, None]
    out_ref[...] += y * w


def _moe_call(x, experts, comb, base):
    grid = (S // _MOE_TILE, E)
    w1, w3, w2 = experts[0], experts[1], experts[2]
    return pl.pallas_call(
        _moe_kernel,
        grid=grid,
        in_specs=[
            pl.BlockSpec((_MOE_TILE, DIM), lambda t, e: (t, 0)),
            pl.BlockSpec((1, DIM, EXP_DIM), lambda t, e: (e, 0, 0)),
            pl.BlockSpec((1, DIM, EXP_DIM), lambda t, e: (e, 0, 0)),
            pl.BlockSpec((1, DIM, EXP_DIM), lambda t, e: (e, 0, 0)),
            pl.BlockSpec((1, 1, _MOE_TILE), lambda t, e: (t * E + e, 0, 0)),
            pl.BlockSpec((_MOE_TILE, DIM), lambda t, e: (t, 0)),
        ],
        out_specs=pl.BlockSpec((_MOE_TILE, DIM), lambda t, e: (t, 0)),
        out_shape=jax.ShapeDtypeStruct((S, DIM), jnp.float32),
    )(x, w1, w3, w2, comb.reshape(-1, 1, _MOE_TILE), base)


def kernel(x_input, p_indices, p_values, f_indices, f_values, attn_w,
           attn_o_w, attn_norm_w, ffn_norm_w, ffn_up_w, ffn_down_w,
           p_ffn_experts, f_ffn_experts, p_token_keys, f_token_keys,
           p_token_router_bias, f_token_router_bias):
    x_flat = x_input.reshape(N, DIM)

    inv_freq = (1.0 / THETA) ** (
        jnp.arange(0, HDIM, 2, dtype=jnp.float32) / HDIM)
    t = jnp.arange(S, dtype=jnp.float32)
    freqs = jnp.outer(t, inv_freq)
    cos_t = jnp.tile(jnp.concatenate([jnp.cos(freqs)] * 2, axis=1),
                     (1, HEADS))
    sin_t = jnp.tile(jnp.concatenate([jnp.sin(freqs)] * 2, axis=1),
                     (1, HEADS))

    w_t = attn_w.T
    wq, wk, wv = w_t[:, :DIM], w_t[:, DIM:2 * DIM], w_t[:, 2 * DIM:]

    def p_rot(w):
        w4 = w.reshape(DIM, HEADS, 2, HDIM // 2)
        return jnp.stack([w4[:, :, 1], -w4[:, :, 0]],
                         axis=2).reshape(DIM, DIM)

    w_aug = jnp.concatenate([wq, wk, wv, p_rot(wq), p_rot(wk)],
                            axis=1).astype(jnp.bfloat16)

    qkv = _qkv_call(x_flat, w_aug, attn_norm_w.reshape(1, DIM), cos_t, sin_t)

    o_flat = _attn_call(qkv)

    resid, x_ffn = _proj_call(o_flat, attn_o_w.T.astype(jnp.bfloat16),
                              x_flat, ffn_norm_w.reshape(1, DIM))

    base = _ffn_call(x_ffn, ffn_up_w.T.astype(jnp.bfloat16),
                     ffn_down_w.T.astype(jnp.bfloat16), resid)

    p_x = x_ffn[:S]
    f_x = x_ffn[S:]

    def side(x_side, idx, vals, keys, bias, experts, base_side):
        idx3 = idx.reshape(S // _RTR_TILE, _RTR_TILE, TOPK)
        vals3 = vals.reshape(S // _RTR_TILE, _RTR_TILE, TOPK)
        comb = _router_call(x_side, keys.astype(jnp.bfloat16), idx3, vals3,
                            bias.reshape(1, E))
        return _moe_call(x_side, experts.astype(jnp.bfloat16), comb,
                         base_side)

    py = side(p_x, p_indices, p_values, p_token_keys,
              p_token_router_bias, p_ffn_experts, base[:S])
    fy = side(f_x, f_indices, f_values, f_token_keys,
              f_token_router_bias, f_ffn_experts, base[S:])

    return jnp.concatenate([py, fy], axis=0).reshape(B, S, DIM)


# speedup vs baseline: 3.5328x; 1.0502x over previous
import functools
import math

import jax
import jax.numpy as jnp
from jax.experimental import pallas as pl

DIM = 768
HEADS = 12
HDIM = 64
E = 8
TOPK = 2
EXP_DIM = 256
DIM_S = 2048
EPS = 1e-5
THETA = 10000.0
B = 2
S = 2048
N = B * S
L = 2 * S


_QKV_TILE = 256


def _qkv_kernel(x_ref, w_ref, nw_ref, cos_ref, sin_ref, out_ref):
    x = x_ref[...]
    xn = x * jax.lax.rsqrt(jnp.mean(x * x, axis=-1, keepdims=True) + EPS)
    xn = (xn * nw_ref[...]).astype(jnp.bfloat16)
    qkv = jnp.dot(xn, w_ref[...], preferred_element_type=jnp.float32)
    cos = cos_ref[...]
    sin = sin_ref[...]
    q = qkv[:, :DIM] * cos + qkv[:, 3 * DIM:4 * DIM] * sin
    k = qkv[:, DIM:2 * DIM] * cos + qkv[:, 4 * DIM:] * sin
    out_ref[...] = jnp.concatenate(
        [q, k, qkv[:, 2 * DIM:3 * DIM]], axis=-1).astype(jnp.bfloat16)


def _qkv_call(x_flat, w_aug, norm_w, cos_t, sin_t):
    grid = (N // _QKV_TILE,)
    n_pos = S // _QKV_TILE
    return pl.pallas_call(
        _qkv_kernel,
        grid=grid,
        in_specs=[
            pl.BlockSpec((_QKV_TILE, DIM), lambda i: (i, 0)),
            pl.BlockSpec((DIM, 5 * DIM), lambda i: (0, 0)),
            pl.BlockSpec((1, DIM), lambda i: (0, 0)),
            pl.BlockSpec((_QKV_TILE, DIM), lambda i: (i % n_pos, 0)),
            pl.BlockSpec((_QKV_TILE, DIM), lambda i: (i % n_pos, 0)),
        ],
        out_specs=pl.BlockSpec((_QKV_TILE, 3 * DIM), lambda i: (i, 0)),
        out_shape=jax.ShapeDtypeStruct((N, 3 * DIM), jnp.bfloat16),
    )(x_flat, w_aug, norm_w, cos_t, sin_t)


_Q_TILE = 512
_K_TILE = 512


def _attn_kernel(q_ref, k_ref, v_ref, out_ref):
    i = pl.program_id(1)
    c = i // (S // _Q_TILE)
    ib = i % (S // _Q_TILE)
    nb = ib + 1
    scale = 1.0 / math.sqrt(HDIM)
    s_row = (jax.lax.broadcasted_iota(jnp.int32, (_Q_TILE, _K_TILE), 0)
             + ib * _Q_TILE)
    lim0 = s_row
    lim1 = s_row - 1 + c
    cols = jax.lax.broadcasted_iota(jnp.int32, (_Q_TILE, _K_TILE), 1)

    def attend(off):
        q = q_ref[:, off:off + HDIM]

        def body(j, carry):
            m, l, acc = carry
            base = j * _K_TILE
            k0 = k_ref[pl.ds(base, _K_TILE), off:off + HDIM]
            k1 = k_ref[pl.ds(S + base, _K_TILE), off:off + HDIM]
            v0 = v_ref[pl.ds(base, _K_TILE), off:off + HDIM]
            v1 = v_ref[pl.ds(S + base, _K_TILE), off:off + HDIM]
            t = cols + base
            l0 = jax.lax.dot_general(
                q, k0, (((1,), (1,)), ((), ())),
                preferred_element_type=jnp.float32) * scale
            l1 = jax.lax.dot_general(
                q, k1, (((1,), (1,)), ((), ())),
                preferred_element_type=jnp.float32) * scale
            l0 = jnp.where(t <= lim0, l0, jnp.float32(-1e30))
            l1 = jnp.where(t <= lim1, l1, jnp.float32(-1e30))
            m_new = jnp.maximum(m, jnp.maximum(
                jnp.max(l0, axis=-1, keepdims=True),
                jnp.max(l1, axis=-1, keepdims=True)))
            p0 = jnp.exp(l0 - m_new)
            p1 = jnp.exp(l1 - m_new)
            alpha = jnp.exp(m - m_new)
            l_new = l * alpha + jnp.sum(p0, axis=-1, keepdims=True) \
                + jnp.sum(p1, axis=-1, keepdims=True)
            acc_new = acc * alpha \
                + jnp.dot(p0.astype(jnp.bfloat16), v0,
                          preferred_element_type=jnp.float32) \
                + jnp.dot(p1.astype(jnp.bfloat16), v1,
                          preferred_element_type=jnp.float32)
            return m_new, l_new, acc_new

        m0 = jnp.full((_Q_TILE, 1), -1e30, dtype=jnp.float32)
        l0_ = jnp.zeros((_Q_TILE, 1), dtype=jnp.float32)
        a0 = jnp.zeros((_Q_TILE, HDIM), dtype=jnp.float32)
        m, l, acc = jax.lax.fori_loop(0, nb, body, (m0, l0_, a0))
        return acc / l

    out_ref[...] = jnp.concatenate(
        [attend(0), attend(HDIM)], axis=1).astype(jnp.bfloat16)


def _attn_call(qkv):
    grid = (HEADS // 2, N // _Q_TILE)
    return pl.pallas_call(
        _attn_kernel,
        grid=grid,
        in_specs=[
            pl.BlockSpec((_Q_TILE, 2 * HDIM), lambda g, i: (i, g)),
            pl.BlockSpec((N, 2 * HDIM), lambda g, i: (0, HEADS // 2 + g)),
            pl.BlockSpec((N, 2 * HDIM), lambda g, i: (0, HEADS + g)),
        ],
        out_specs=pl.BlockSpec((_Q_TILE, 2 * HDIM), lambda g, i: (i, g)),
        out_shape=jax.ShapeDtypeStruct((N, DIM), jnp.bfloat16),
    )(qkv, qkv, qkv)


_PF_TILE = 512


def _projffn_kernel(o_ref, wo_ref, x_ref, nw_ref, up_ref, down_ref,
                    xffn_ref, base_ref):
    y = jnp.dot(o_ref[...], wo_ref[...], preferred_element_type=jnp.float32)
    resid = y + x_ref[...]
    xn = resid * jax.lax.rsqrt(
        jnp.mean(resid * resid, axis=-1, keepdims=True) + EPS)
    xf = (xn * nw_ref[...]).astype(jnp.bfloat16)
    xffn_ref[...] = xf
    h = jnp.dot(xf, up_ref[...], preferred_element_type=jnp.float32)
    x1 = h[:, :DIM_S]
    x2 = h[:, DIM_S:]
    g = ((x1 * jax.lax.logistic(x1)) * x2).astype(jnp.bfloat16)
    ys = jnp.dot(g, down_ref[...], preferred_element_type=jnp.float32)
    base_ref[...] = ys + resid


def _projffn_call(o_flat, wo_t, x_flat, norm_w, up_t, down_t):
    grid = (N // _PF_TILE,)
    return pl.pallas_call(
        _projffn_kernel,
        grid=grid,
        in_specs=[
            pl.BlockSpec((_PF_TILE, DIM), lambda i: (i, 0)),
            pl.BlockSpec((DIM, DIM), lambda i: (0, 0)),
            pl.BlockSpec((_PF_TILE, DIM), lambda i: (i, 0)),
            pl.BlockSpec((1, DIM), lambda i: (0, 0)),
            pl.BlockSpec((DIM, 2 * DIM_S), lambda i: (0, 0)),
            pl.BlockSpec((DIM_S, DIM), lambda i: (0, 0)),
        ],
        out_specs=[
            pl.BlockSpec((_PF_TILE, DIM), lambda i: (i, 0)),
            pl.BlockSpec((_PF_TILE, DIM), lambda i: (i, 0)),
        ],
        out_shape=[
            jax.ShapeDtypeStruct((N, DIM), jnp.bfloat16),
            jax.ShapeDtypeStruct((N, DIM), jnp.float32),
        ],
    )(o_flat, wo_t, x_flat, norm_w, up_t, down_t)


_MOE_TILE = 512


def _moe_kernel(x_ref, keys_ref, idx_ref, vals_ref, bias_ref,
                w1_ref, w3_ref, w2_ref, base_ref, out_ref):
    x = x_ref[...]
    tok = jnp.dot(x, keys_ref[...], preferred_element_type=jnp.float32)
    idx = idx_ref[0]
    onehot = (idx[:, :, None] ==
              jnp.arange(E, dtype=idx.dtype)[None, None, :]).astype(
                  jnp.float32)
    gathered = jnp.sum(onehot * tok[:, None, :], axis=-1)
    gbias = jnp.sum(onehot * bias_ref[...][None, :, :], axis=-1)
    v = vals_ref[0] + gathered + gbias
    sc = jax.lax.logistic(v)
    sc = sc / jnp.sum(sc, axis=-1, keepdims=True)
    comb = jnp.sum(onehot * sc[:, :, None], axis=1)

    out_ref[...] = base_ref[...]
    for e in range(E):
        h1 = jnp.dot(x, w1_ref[e], preferred_element_type=jnp.float32)
        h3 = jnp.dot(x, w3_ref[e], preferred_element_type=jnp.float32)
        h = ((h1 * jax.lax.logistic(h1)) * h3).astype(jnp.bfloat16)
        y = jax.lax.dot_general(h, w2_ref[e], (((1,), (1,)), ((), ())),
                                preferred_element_type=jnp.float32)
        out_ref[...] += y * comb[:, e:e + 1]


def _moe_call(x_side, keys, idx, vals, bias, experts, base_side):
    grid = (S // _MOE_TILE,)
    nt = S // _MOE_TILE
    w1, w3, w2 = experts[0], experts[1], experts[2]
    idx3 = idx.reshape(nt, _MOE_TILE, TOPK)
    vals3 = vals.reshape(nt, _MOE_TILE, TOPK)
    return pl.pallas_call(
        _moe_kernel,
        grid=grid,
        in_specs=[
            pl.BlockSpec((_MOE_TILE, DIM), lambda i: (i, 0)),
            pl.BlockSpec((DIM, E), lambda i: (0, 0)),
            pl.BlockSpec((1, _MOE_TILE, TOPK), lambda i: (i, 0, 0)),
            pl.BlockSpec((1, _MOE_TILE, TOPK), lambda i: (i, 0, 0)),
            pl.BlockSpec((1, E), lambda i: (0, 0)),
            pl.BlockSpec((E, DIM, EXP_DIM), lambda i: (0, 0, 0)),
            pl.BlockSpec((E, DIM, EXP_DIM), lambda i: (0, 0, 0)),
            pl.BlockSpec((E, DIM, EXP_DIM), lambda i: (0, 0, 0)),
            pl.BlockSpec((_MOE_TILE, DIM), lambda i: (i, 0)),
        ],
        out_specs=pl.BlockSpec((_MOE_TILE, DIM), lambda i: (i, 0)),
        out_shape=jax.ShapeDtypeStruct((S, DIM), jnp.float32),
    )(x_side, keys, idx3, vals3, bias, w1, w3, w2, base_side)


def kernel(x_input, p_indices, p_values, f_indices, f_values, attn_w,
           attn_o_w, attn_norm_w, ffn_norm_w, ffn_up_w, ffn_down_w,
           p_ffn_experts, f_ffn_experts, p_token_keys, f_token_keys,
           p_token_router_bias, f_token_router_bias):
    x_flat = x_input.reshape(N, DIM)

    inv_freq = (1.0 / THETA) ** (
        jnp.arange(0, HDIM, 2, dtype=jnp.float32) / HDIM)
    t = jnp.arange(S, dtype=jnp.float32)
    freqs = jnp.outer(t, inv_freq)
    cos_t = jnp.tile(jnp.concatenate([jnp.cos(freqs)] * 2, axis=1),
                     (1, HEADS))
    sin_t = jnp.tile(jnp.concatenate([jnp.sin(freqs)] * 2, axis=1),
                     (1, HEADS))

    w_t = attn_w.T
    wq, wk, wv = w_t[:, :DIM], w_t[:, DIM:2 * DIM], w_t[:, 2 * DIM:]

    def p_rot(w):
        w4 = w.reshape(DIM, HEADS, 2, HDIM // 2)
        return jnp.stack([w4[:, :, 1], -w4[:, :, 0]],
                         axis=2).reshape(DIM, DIM)

    w_aug = jnp.concatenate([wq, wk, wv, p_rot(wq), p_rot(wk)],
                            axis=1).astype(jnp.bfloat16)

    qkv = _qkv_call(x_flat, w_aug, attn_norm_w.reshape(1, DIM), cos_t, sin_t)

    o_flat = _attn_call(qkv)

    x_ffn, base = _projffn_call(
        o_flat, attn_o_w.T.astype(jnp.bfloat16), x_flat,
        ffn_norm_w.reshape(1, DIM), ffn_up_w.T.astype(jnp.bfloat16),
        ffn_down_w.T.astype(jnp.bfloat16))

    def side(x_side, idx, vals, keys, bias, experts, base_side):
        return _moe_call(x_side, keys.astype(jnp.bfloat16), idx, vals,
                         bias.reshape(1, E), experts.astype(jnp.bfloat16),
                         base_side)

    py = side(x_ffn[:S], p_indices, p_values, p_token_keys,
              p_token_router_bias, p_ffn_experts, base[:S])
    fy = side(x_ffn[S:], f_indices, f_values, f_token_keys,
              f_token_router_bias, f_ffn_experts, base[S:])
    return jnp.concatenate([py, fy], axis=0).reshape(B, S, DIM)
